# baseline (speedup 1.0000x reference)
import functools

import numpy as np
import jax
import jax.numpy as jnp
from jax import lax
from jax.experimental import pallas as pl
from jax.experimental.pallas import tpu as pltpu
from jax.experimental.pallas import tpu_sc as plsc

_DIM = 768
_SEQ = 64
_NC = 2
_NS = 16
_NW = _NC * _NS
_CH = 32
_NG = _DIM // 16


def _pos_table() -> np.ndarray:
    pos = np.arange(_SEQ, dtype=np.float32)[:, None]
    i = np.arange(_DIM // 2, dtype=np.float32)[None, :]
    angle = pos / np.power(10000.0, 2.0 * i / _DIM)
    return np.concatenate([np.sin(angle), np.cos(angle)], axis=-1)


def _build(total_tokens: int):
    t_per_w = total_tokens // _NW
    n_ch = t_per_w // _CH
    mesh = plsc.VectorSubcoreMesh(
        core_axis_name="c", subcore_axis_name="s",
        num_cores=_NC, num_subcores=_NS,
    )

    @functools.partial(
        pl.kernel,
        mesh=mesh,
        out_type=jax.ShapeDtypeStruct((total_tokens, _DIM), jnp.float32),
        scratch_types=[
            pltpu.VMEM((t_per_w,), jnp.int32),
            pltpu.VMEM((_SEQ, _DIM), jnp.float32),
            [pltpu.VMEM((_CH, _DIM), jnp.float32) for _ in range(2)],
            [pltpu.SemaphoreType.DMA for _ in range(2)],
        ],
    )
    def run(idx_hbm, wte_hbm, pos_hbm, out_hbm, idx_v, pos_v, rows, sems):
        wid = lax.axis_index("s") * _NC + lax.axis_index("c")
        base = wid * t_per_w

        pltpu.sync_copy(idx_hbm.at[pl.ds(base, t_per_w)], idx_v)
        pltpu.sync_copy(pos_hbm, pos_v)

        def gather_desc(g, s):
            return pltpu.make_async_copy(
                wte_hbm.at[idx_v.at[pl.ds(g * _CH, _CH)]], rows[s], sems[s])

        def scatter_desc(g, s):
            return pltpu.make_async_copy(
                rows[s], out_hbm.at[pl.ds(base + g * _CH, _CH)], sems[s])

        gather_desc(0, 0).start()

        def body(it, carry):
            for k in range(2):
                g = 2 * it + k

                @pl.when(g >= 1)
                def _():
                    scatter_desc(g - 1, 1 - k).wait()

                @pl.when(g + 1 < n_ch)
                def _():
                    gather_desc(g + 1, 1 - k).start()

                gather_desc(g, k).wait()

                def add_row(r, c):
                    for cg in range(_NG):
                        sl = pl.ds(cg * 16, 16)
                        rows[k][r, sl] = rows[k][r, sl] + pos_v[k * _CH + r, sl]
                    return c

                lax.fori_loop(0, _CH, add_row, 0)
                scatter_desc(g, k).start()
            return carry

        lax.fori_loop(0, n_ch // 2, body, 0)
        scatter_desc(n_ch - 1, 1).wait()

    return run


def kernel(x, wte):
    b, s = x.shape
    total = b * s
    idx_flat = x.reshape(total).astype(jnp.int32)
    pos = jnp.asarray(_pos_table())
    out = _build(total)(idx_flat, wte, pos)
    return out.reshape(b, s, _DIM)


# speedup vs baseline: 1.0900x; 1.0900x over previous
import functools

import numpy as np
import jax
import jax.numpy as jnp
from jax import lax
from jax.experimental import pallas as pl
from jax.experimental.pallas import tpu as pltpu
from jax.experimental.pallas import tpu_sc as plsc

_DIM = 768
_SEQ = 64
_NC = 2
_NS = 16
_NW = _NC * _NS
_CH = 16
_NSLOT = 4
_NG = _DIM // 16


def _pos_table() -> np.ndarray:
    pos = np.arange(_SEQ, dtype=np.float32)[:, None]
    i = np.arange(_DIM // 2, dtype=np.float32)[None, :]
    angle = pos / np.power(10000.0, 2.0 * i / _DIM)
    return np.concatenate([np.sin(angle), np.cos(angle)], axis=-1)


def _build(total_tokens: int):
    t_per_w = total_tokens // _NW
    n_ch = t_per_w // _CH
    mesh = plsc.VectorSubcoreMesh(
        core_axis_name="c", subcore_axis_name="s",
        num_cores=_NC, num_subcores=_NS,
    )

    @functools.partial(
        pl.kernel,
        mesh=mesh,
        out_type=jax.ShapeDtypeStruct((total_tokens, _DIM), jnp.float32),
        scratch_types=[
            pltpu.VMEM((t_per_w,), jnp.int32),
            pltpu.VMEM((_SEQ, _DIM), jnp.float32),
            [pltpu.VMEM((_CH, _DIM), jnp.float32) for _ in range(_NSLOT)],
            [pltpu.SemaphoreType.DMA for _ in range(_NSLOT)],
        ],
    )
    def run(idx_hbm, wte_hbm, pos_hbm, out_hbm, idx_v, pos_v, rows, sems):
        wid = lax.axis_index("s") * _NC + lax.axis_index("c")
        base = wid * t_per_w

        pltpu.sync_copy(idx_hbm.at[pl.ds(base, t_per_w)], idx_v)
        pltpu.sync_copy(pos_hbm, pos_v)

        def gather_desc(g, s):
            return pltpu.make_async_copy(
                wte_hbm.at[idx_v.at[pl.ds(g * _CH, _CH)]], rows[s], sems[s])

        def scatter_desc(g, s):
            return pltpu.make_async_copy(
                rows[s], out_hbm.at[pl.ds(base + g * _CH, _CH)], sems[s])

        def body(it, carry):
            for j in range(_NSLOT):
                g = it * _NSLOT + j
                gj = g - 2
                jj = (j - 2) % _NSLOT

                @pl.when(jnp.logical_and(g >= _NSLOT, g - _NSLOT < n_ch))
                def _():
                    scatter_desc(g - _NSLOT, j).wait()

                @pl.when(g < n_ch)
                def _():
                    gather_desc(g, j).start()

                @pl.when(jnp.logical_and(gj >= 0, gj < n_ch))
                def _():
                    gather_desc(gj, jj).wait()

                    def add_row(r, c):
                        for cg in range(_NG):
                            sl = pl.ds(cg * 16, 16)
                            rows[jj][r, sl] = (rows[jj][r, sl]
                                               + pos_v[jj * _CH + r, sl])
                        return c

                    lax.fori_loop(0, _CH, add_row, 0)
                    scatter_desc(gj, jj).start()
            return carry

        lax.fori_loop(0, n_ch // _NSLOT + 1, body, 0)

    return run


def kernel(x, wte):
    b, s = x.shape
    total = b * s
    idx_flat = x.reshape(total).astype(jnp.int32)
    pos = jnp.asarray(_pos_table())
    out = _build(total)(idx_flat, wte, pos)
    return out.reshape(b, s, _DIM)
